# baseline (speedup 1.0000x reference)
import functools

import jax
import jax.numpy as jnp
from jax import lax
from jax.experimental import pallas as pl
from jax.experimental.pallas import tpu as pltpu
from jax.experimental.pallas import tpu_sc as plsc

_SC_LANES = 16


def _make_keep_sc(n: int):
    mesh = plsc.VectorSubcoreMesh(core_axis_name="c", subcore_axis_name="s")

    @functools.partial(
        pl.kernel,
        mesh=mesh,
        out_type=jax.ShapeDtypeStruct((n,), jnp.int32),
        scratch_types=[
            pltpu.VMEM((n,), jnp.int32),
            pltpu.VMEM((_SC_LANES,), jnp.int32),
            pltpu.VMEM((n,), jnp.int32),
            pltpu.SemaphoreType.DMA,
        ],
    )
    def keep_sc(s_hbm, kn_hbm, keep_hbm, s_v, kn_v, vals_v, sem):
        wid = lax.axis_index("s") * 2 + lax.axis_index("c")

        @pl.when(wid == 0)
        def _():
            pltpu.sync_copy(s_hbm, s_v)
            pltpu.sync_copy(kn_hbm, kn_v)
            kn_vec = kn_v[...]
            base = lax.iota(jnp.int32, 16)
            one = jnp.full((_SC_LANES,), 1, jnp.int32)
            zero = jnp.full((_SC_LANES,), 0, jnp.int32)
            for i in range(n // _SC_LANES):
                cur = base + (i * _SC_LANES)
                vals_v[pl.ds(i * _SC_LANES, _SC_LANES)] = jnp.where(
                    cur < kn_vec, one, zero)
            pltpu.async_copy(vals_v, keep_hbm.at[s_v], sem).wait()

    return keep_sc


def _select_kernel(bm_ref, keep_ref, emb_ref, mt_ref, out_ref):
    tb = pl.program_id(1)
    t_blk = out_ref.shape[1]
    keep_blk = keep_ref[pl.ds(tb * t_blk, t_blk), :]
    out_ref[...] = jnp.where(keep_blk[None, :, :] != 0, emb_ref[...],
                             mt_ref[0, 0][None, None, :])


def kernel(embeddings, mask_token, shuffled_indices, skip):
    B, N, D = embeddings.shape
    n = shuffled_indices.shape[0]
    T = 256 if n % 256 == 0 else n
    TB = n // T
    BB = 16 if B % 16 == 0 else (4 if B % 4 == 0 else 1)

    keep_n = jnp.asarray(n - skip, dtype=jnp.int32)
    s_i32 = shuffled_indices.astype(jnp.int32)
    kn16 = jnp.full((_SC_LANES,), keep_n, jnp.int32)

    keep = _make_keep_sc(n)(s_i32, kn16)
    keep2d = keep.reshape(n, 1)

    idx = jnp.arange(n, dtype=jnp.int32)
    in_blk = s_i32 // T
    is_kept = (idx < keep_n).astype(jnp.int32)
    counts = jnp.sum(
        jnp.where(in_blk[:, None] == jnp.arange(TB, dtype=jnp.int32)[None, :],
                  is_kept[:, None], 0),
        axis=0)
    bm = lax.cummax(jnp.where(counts > 0, jnp.arange(TB, dtype=jnp.int32), 0))

    grid_spec = pltpu.PrefetchScalarGridSpec(
        num_scalar_prefetch=1,
        grid=(B // BB, TB),
        in_specs=[
            pl.BlockSpec((n, 1), lambda b, tb, bm: (0, 0)),
            pl.BlockSpec((BB, T, D), lambda b, tb, bm: (b, bm[tb], 0)),
            pl.BlockSpec((1, 1, D), lambda b, tb, bm: (0, 0, 0)),
        ],
        out_specs=pl.BlockSpec((BB, T, D), lambda b, tb, bm: (b, tb, 0)),
    )

    return pl.pallas_call(
        _select_kernel,
        grid_spec=grid_spec,
        out_shape=jax.ShapeDtypeStruct((B, N, D), embeddings.dtype),
    )(bm, keep2d, embeddings, mask_token)


# speedup vs baseline: 1.8642x; 1.8642x over previous
import jax
import jax.numpy as jnp
from jax.experimental import pallas as pl
from jax.experimental.pallas import tpu as pltpu


def _mask_kernel(bm_ref, kn_ref, s_ref, emb_ref, mt_ref, out_ref, keep_ref):
    b = pl.program_id(0)
    tb = pl.program_id(1)
    n = keep_ref.shape[0]

    @pl.when(jnp.logical_and(b == 0, tb == 0))
    def _compute_keep():
        s_row = s_ref[...]
        i_row = jax.lax.broadcasted_iota(jnp.int32, (1, n), 1)
        valid = (i_row < kn_ref[0]).astype(jnp.int32)
        t_col = jax.lax.broadcasted_iota(jnp.int32, (n, 1), 0)
        hit = jnp.where(s_row == t_col, valid, 0)
        keep_ref[...] = jnp.max(hit, axis=1, keepdims=True)

    t_blk = out_ref.shape[1]
    keep_blk = keep_ref[pl.ds(tb * t_blk, t_blk), :]
    out_ref[...] = jnp.where(keep_blk[None, :, :] != 0, emb_ref[...],
                             mt_ref[0, 0][None, None, :])


def kernel(embeddings, mask_token, shuffled_indices, skip):
    B, N, D = embeddings.shape
    n = shuffled_indices.shape[0]
    T = 256 if n % 256 == 0 else n
    TB = n // T
    BB = 16 if B % 16 == 0 else (4 if B % 4 == 0 else 1)

    keep_n = jnp.asarray(n - skip, dtype=jnp.int32).reshape(1)
    s2d = shuffled_indices.astype(jnp.int32).reshape(1, n)

    idx = jnp.arange(n, dtype=jnp.int32)
    in_blk = shuffled_indices.astype(jnp.int32) // T
    is_kept = (idx < keep_n[0]).astype(jnp.int32)
    counts = jnp.sum(
        jnp.where(in_blk[:, None] == jnp.arange(TB, dtype=jnp.int32)[None, :],
                  is_kept[:, None], 0),
        axis=0)
    bm = jax.lax.cummax(jnp.where(counts > 0, jnp.arange(TB, dtype=jnp.int32), 0))

    grid_spec = pltpu.PrefetchScalarGridSpec(
        num_scalar_prefetch=2,
        grid=(B // BB, TB),
        in_specs=[
            pl.BlockSpec((1, n), lambda b, tb, bm, kn: (0, 0)),
            pl.BlockSpec((BB, T, D), lambda b, tb, bm, kn: (b, bm[tb], 0)),
            pl.BlockSpec((1, 1, D), lambda b, tb, bm, kn: (0, 0, 0)),
        ],
        out_specs=pl.BlockSpec((BB, T, D), lambda b, tb, bm, kn: (b, tb, 0)),
        scratch_shapes=[pltpu.VMEM((n, 1), jnp.int32)],
    )

    return pl.pallas_call(
        _mask_kernel,
        grid_spec=grid_spec,
        out_shape=jax.ShapeDtypeStruct((B, N, D), embeddings.dtype),
    )(bm, keep_n, s2d, embeddings, mask_token)
